# Initial kernel scaffold; baseline (speedup 1.0000x reference)
import functools
import math

import numpy as np
import jax
import jax.numpy as jnp
from jax import lax
from jax.experimental import pallas as pl
from jax.experimental.pallas import tpu as pltpu

_E_LAYERS = 2
_D = 1024
_H = 16
_DH = 64
_DFF = 4096
_L = 2048
_FACTOR = 5
_SAMPLE_K = min(_FACTOR * int(math.ceil(math.log(_L))), _L)
_NTOP = min(_FACTOR * int(math.ceil(math.log(_L))), _L)

_PREC_HI = lax.Precision.HIGHEST


def _count_matrix(layer: int) -> np.ndarray:
    idx = np.asarray(
        jax.random.randint(jax.random.key(1234 + layer), (_L, _SAMPLE_K), 0, _L)
    )
    c = np.zeros((_L, _L), np.int8)
    np.add.at(c, (np.arange(_L)[:, None], idx), 1)
    return c


_COUNTS = [_count_matrix(i) for i in range(_E_LAYERS)]


def _ln(z, g, b):
    m = jnp.mean(z, axis=-1, keepdims=True)
    c = z - m
    v = jnp.mean(c * c, axis=-1, keepdims=True)
    return c * lax.rsqrt(v + 1e-5) * g + b


def _mm_bias_body(x_ref, w_ref, b_ref, o_ref, *, relu):
    acc = jnp.dot(x_ref[...], w_ref[...], preferred_element_type=jnp.float32,
                  precision=_PREC_HI)
    acc = acc + b_ref[...]
    if relu:
        acc = jnp.maximum(acc, 0.0)
    o_ref[...] = acc


def _mm_bias(x, w, b, relu=False, bm=512, bn=1024):
    m, k = x.shape
    n = w.shape[1]
    body = functools.partial(_mm_bias_body, relu=relu)
    return pl.pallas_call(
        body,
        grid=(n // bn, m // bm),
        in_specs=[
            pl.BlockSpec((bm, k), lambda j, i: (i, 0)),
            pl.BlockSpec((k, bn), lambda j, i: (0, j)),
            pl.BlockSpec((1, bn), lambda j, i: (0, j)),
        ],
        out_specs=pl.BlockSpec((bm, bn), lambda j, i: (i, j)),
        out_shape=jax.ShapeDtypeStruct((m, n), jnp.float32),
    )(x, w, b.reshape(1, n))


def _mm_res_ln_body(y_ref, w_ref, b_ref, r_ref, g_ref, be_ref, o_ref):
    z = jnp.dot(y_ref[...], w_ref[...], preferred_element_type=jnp.float32,
                precision=_PREC_HI)
    z = z + b_ref[...] + r_ref[...]
    o_ref[...] = _ln(z, g_ref[...], be_ref[...])


def _mm_res_ln_final_body(y_ref, w_ref, b_ref, r_ref, g_ref, be_ref,
                          gf_ref, bf_ref, o_ref):
    z = jnp.dot(y_ref[...], w_ref[...], preferred_element_type=jnp.float32,
                precision=_PREC_HI)
    z = z + b_ref[...] + r_ref[...]
    z = _ln(z, g_ref[...], be_ref[...])
    o_ref[...] = _ln(z, gf_ref[...], bf_ref[...])


def _mm_res_ln(y, w, b, res, g, be, final=None, bm=512):
    m, k = y.shape
    n = w.shape[1]
    args = [y, w, b.reshape(1, n), res, g.reshape(1, n), be.reshape(1, n)]
    in_specs = [
        pl.BlockSpec((bm, k), lambda i: (i, 0)),
        pl.BlockSpec((k, n), lambda i: (0, 0)),
        pl.BlockSpec((1, n), lambda i: (0, 0)),
        pl.BlockSpec((bm, n), lambda i: (i, 0)),
        pl.BlockSpec((1, n), lambda i: (0, 0)),
        pl.BlockSpec((1, n), lambda i: (0, 0)),
    ]
    if final is None:
        body = _mm_res_ln_body
    else:
        body = _mm_res_ln_final_body
        gf, bf = final
        args += [gf.reshape(1, n), bf.reshape(1, n)]
        in_specs += [pl.BlockSpec((1, n), lambda i: (0, 0)),
                     pl.BlockSpec((1, n), lambda i: (0, 0))]
    return pl.pallas_call(
        body,
        grid=(m // bm,),
        in_specs=in_specs,
        out_specs=pl.BlockSpec((bm, n), lambda i: (i, 0)),
        out_shape=jax.ShapeDtypeStruct((m, n), jnp.float32),
    )(*args)


def _attn_body(c_ref, q_ref, k_ref, v_ref, o_ref):
    Q = q_ref[...]
    K = k_ref[...]
    V = v_ref[...]
    iota_l = lax.broadcasted_iota(jnp.int32, (1, _L), 1)

    chunks = []
    cs = _L // 4
    for t in range(4):
        qt = Q[t * cs:(t + 1) * cs]
        qk = lax.dot_general(qt, K, (((1,), (1,)), ((), ())),
                             preferred_element_type=jnp.float32,
                             precision=_PREC_HI)
        ct = c_ref[t * cs:(t + 1) * cs].astype(jnp.float32)
        mx = jnp.max(jnp.where(ct > 0.0, qk, -jnp.inf), axis=1)
        sm = jnp.sum(qk * ct, axis=1) * (1.0 / _SAMPLE_K)
        chunks.append((mx - sm).reshape(1, cs))
    M = jnp.concatenate(chunks, axis=1)

    sub_iota = lax.broadcasted_iota(jnp.int32, (_NTOP, 1), 0)

    def step(i, carry):
        Mc, oh = carry
        amax = jnp.max(Mc)
        idx = jnp.min(jnp.where(Mc == amax, iota_l, _L))
        oh = jnp.where((sub_iota == i) & (iota_l == idx), 1.0, oh)
        Mc = jnp.where(iota_l == idx, -jnp.inf, Mc)
        return Mc, oh

    _, onehot = lax.fori_loop(
        0, _NTOP, step, (M, jnp.zeros((_NTOP, _L), jnp.float32)))

    Qr = lax.dot_general(onehot, Q, (((1,), (0,)), ((), ())),
                         preferred_element_type=jnp.float32,
                         precision=_PREC_HI)
    scores = lax.dot_general(Qr, K, (((1,), (1,)), ((), ())),
                             preferred_element_type=jnp.float32,
                             precision=_PREC_HI) * (1.0 / math.sqrt(_DH))
    smax = jnp.max(scores, axis=1, keepdims=True)
    p = jnp.exp(scores - smax)
    attn = p / jnp.sum(p, axis=1, keepdims=True)
    out_top = lax.dot_general(attn, V, (((1,), (0,)), ((), ())),
                              preferred_element_type=jnp.float32,
                              precision=_PREC_HI)
    meanV = jnp.mean(V, axis=0, keepdims=True)
    scat = lax.dot_general(onehot, out_top, (((0,), (0,)), ((), ())),
                           preferred_element_type=jnp.float32,
                           precision=_PREC_HI)
    rowsel = lax.dot_general(onehot, jnp.ones((_NTOP, 1), jnp.float32),
                             (((0,), (0,)), ((), ())),
                             preferred_element_type=jnp.float32,
                             precision=_PREC_HI)
    o_ref[...] = scat + (1.0 - rowsel) * meanV


def _attn(qkv, counts):
    return pl.pallas_call(
        _attn_body,
        grid=(_H,),
        in_specs=[
            pl.BlockSpec((_L, _L), lambda h: (0, 0)),
            pl.BlockSpec((_L, _DH), lambda h: (0, h)),
            pl.BlockSpec((_L, _DH), lambda h: (0, _H + h)),
            pl.BlockSpec((_L, _DH), lambda h: (0, 2 * _H + h)),
        ],
        out_specs=pl.BlockSpec((_L, _DH), lambda h: (0, h)),
        out_shape=jax.ShapeDtypeStruct((_L, _D), jnp.float32),
    )(counts, qkv, qkv, qkv)


def kernel(x, Wq, bq, Wk, bk, Wv, bv, Wo, bo, W1, b1, W2, b2,
           g1, be1, g2, be2, gF, bF):
    xb = x[0]
    for i in range(_E_LAYERS):
        wqkv = jnp.concatenate([Wq[i], Wk[i], Wv[i]], axis=1)
        bqkv = jnp.concatenate([bq[i], bk[i], bv[i]])
        qkv = _mm_bias(xb, wqkv, bqkv)
        ctx = _attn(qkv, jnp.asarray(_COUNTS[i]))
        xb = _mm_res_ln(ctx, Wo[i], bo[i], xb, g1[i], be1[i])
        h = _mm_bias(xb, W1[i], b1[i], relu=True)
        final = (gF, bF) if i == _E_LAYERS - 1 else None
        xb = _mm_res_ln(h, W2[i], b2[i], xb, g2[i], be2[i], final=final)
    return xb[None]


# speedup vs baseline: 1.3042x; 1.3042x over previous
import functools
import math

import numpy as np
import jax
import jax.numpy as jnp
from jax import lax
from jax.experimental import pallas as pl
from jax.experimental.pallas import tpu as pltpu

_E_LAYERS = 2
_D = 1024
_H = 16
_DH = 64
_DFF = 4096
_L = 2048
_FACTOR = 5
_SAMPLE_K = min(_FACTOR * int(math.ceil(math.log(_L))), _L)
_NTOP = min(_FACTOR * int(math.ceil(math.log(_L))), _L)

_PREC_HI = lax.Precision.HIGHEST


def _count_matrix(layer: int) -> np.ndarray:
    idx = np.asarray(
        jax.random.randint(jax.random.key(1234 + layer), (_L, _SAMPLE_K), 0, _L)
    )
    c = np.zeros((_L, _L), np.int8)
    np.add.at(c, (np.arange(_L)[:, None], idx), 1)
    return c


_COUNTS = [_count_matrix(i) for i in range(_E_LAYERS)]


def _ln(z, g, b):
    m = jnp.mean(z, axis=-1, keepdims=True)
    c = z - m
    v = jnp.mean(c * c, axis=-1, keepdims=True)
    return c * lax.rsqrt(v + 1e-5) * g + b


def _mm_bias_body(x_ref, w_ref, b_ref, o_ref, *, relu):
    acc = jnp.dot(x_ref[...], w_ref[...], preferred_element_type=jnp.float32,
                  precision=_PREC_HI)
    acc = acc + b_ref[...]
    if relu:
        acc = jnp.maximum(acc, 0.0)
    o_ref[...] = acc


def _mm_bias(x, w, b, relu=False, bm=512, bn=1024):
    m, k = x.shape
    n = w.shape[1]
    body = functools.partial(_mm_bias_body, relu=relu)
    return pl.pallas_call(
        body,
        grid=(n // bn, m // bm),
        in_specs=[
            pl.BlockSpec((bm, k), lambda j, i: (i, 0)),
            pl.BlockSpec((k, bn), lambda j, i: (0, j)),
            pl.BlockSpec((1, bn), lambda j, i: (0, j)),
        ],
        out_specs=pl.BlockSpec((bm, bn), lambda j, i: (i, j)),
        out_shape=jax.ShapeDtypeStruct((m, n), jnp.float32),
    )(x, w, b.reshape(1, n))


def _mm_res_ln_body(y_ref, w_ref, b_ref, r_ref, g_ref, be_ref, o_ref,
                    acc_ref, *, nk):
    kk = pl.program_id(1)

    @pl.when(kk == 0)
    def _():
        acc_ref[...] = jnp.zeros_like(acc_ref)

    acc_ref[...] += jnp.dot(y_ref[...], w_ref[...],
                            preferred_element_type=jnp.float32,
                            precision=_PREC_HI)

    @pl.when(kk == nk - 1)
    def _():
        z = acc_ref[...] + b_ref[...] + r_ref[...]
        o_ref[...] = _ln(z, g_ref[...], be_ref[...])


def _mm_res_ln(y, w, b, res, g, be, final=None, bm=512, bk=1024):
    m, k = y.shape
    n = w.shape[1]
    nk = k // bk
    if final is not None:
        gf, bf = final
        body = functools.partial(_mm_res_ln_final_body, nk=nk)
        args = [y, w, b.reshape(1, n), res, g.reshape(1, n),
                be.reshape(1, n), gf.reshape(1, n), bf.reshape(1, n)]
        extra = [pl.BlockSpec((1, n), lambda i, kk: (0, 0)),
                 pl.BlockSpec((1, n), lambda i, kk: (0, 0))]
    else:
        body = functools.partial(_mm_res_ln_body, nk=nk)
        args = [y, w, b.reshape(1, n), res, g.reshape(1, n), be.reshape(1, n)]
        extra = []
    in_specs = [
        pl.BlockSpec((bm, bk), lambda i, kk: (i, kk)),
        pl.BlockSpec((bk, n), lambda i, kk: (kk, 0)),
        pl.BlockSpec((1, n), lambda i, kk: (0, 0)),
        pl.BlockSpec((bm, n), lambda i, kk: (i, 0)),
        pl.BlockSpec((1, n), lambda i, kk: (0, 0)),
        pl.BlockSpec((1, n), lambda i, kk: (0, 0)),
    ] + extra
    return pl.pallas_call(
        body,
        grid=(m // bm, nk),
        in_specs=in_specs,
        out_specs=pl.BlockSpec((bm, n), lambda i, kk: (i, 0)),
        out_shape=jax.ShapeDtypeStruct((m, n), jnp.float32),
        scratch_shapes=[pltpu.VMEM((bm, n), jnp.float32)],
    )(*args)


def _mm_res_ln_final_body(y_ref, w_ref, b_ref, r_ref, g_ref, be_ref,
                          gf_ref, bf_ref, o_ref, acc_ref, *, nk):
    kk = pl.program_id(1)

    @pl.when(kk == 0)
    def _():
        acc_ref[...] = jnp.zeros_like(acc_ref)

    acc_ref[...] += jnp.dot(y_ref[...], w_ref[...],
                            preferred_element_type=jnp.float32,
                            precision=_PREC_HI)

    @pl.when(kk == nk - 1)
    def _():
        z = acc_ref[...] + b_ref[...] + r_ref[...]
        z = _ln(z, g_ref[...], be_ref[...])
        o_ref[...] = _ln(z, gf_ref[...], bf_ref[...])


def _attn_body(c_ref, q_ref, k_ref, v_ref, o_ref):
    for s in range(2):
        sl = slice(s * _DH, (s + 1) * _DH)
        o_ref[:, sl] = _one_head(c_ref, q_ref[:, sl], k_ref[:, sl],
                                 v_ref[:, sl])


def _one_head(c_ref, Q, K, V):
    iota_l = lax.broadcasted_iota(jnp.int32, (1, _L), 1)

    chunks = []
    cs = _L // 4
    for t in range(4):
        qt = Q[t * cs:(t + 1) * cs]
        qk = lax.dot_general(qt, K, (((1,), (1,)), ((), ())),
                             preferred_element_type=jnp.float32,
                             precision=_PREC_HI)
        ct = c_ref[t * cs:(t + 1) * cs].astype(jnp.float32)
        mx = jnp.max(jnp.where(ct > 0.0, qk, -jnp.inf), axis=1)
        sm = jnp.sum(qk * ct, axis=1) * (1.0 / _SAMPLE_K)
        chunks.append((mx - sm).reshape(1, cs))
    M = jnp.concatenate(chunks, axis=1)

    sub_iota = lax.broadcasted_iota(jnp.int32, (_NTOP, 1), 0)

    def step(i, carry):
        Mc, oh = carry
        amax = jnp.max(Mc)
        idx = jnp.min(jnp.where(Mc == amax, iota_l, _L))
        oh = jnp.where((sub_iota == i) & (iota_l == idx), 1.0, oh)
        Mc = jnp.where(iota_l == idx, -jnp.inf, Mc)
        return Mc, oh

    _, onehot = lax.fori_loop(
        0, _NTOP, step, (M, jnp.zeros((_NTOP, _L), jnp.float32)))

    Qr = lax.dot_general(onehot, Q, (((1,), (0,)), ((), ())),
                         preferred_element_type=jnp.float32,
                         precision=_PREC_HI)
    scores = lax.dot_general(Qr, K, (((1,), (1,)), ((), ())),
                             preferred_element_type=jnp.float32,
                             precision=_PREC_HI) * (1.0 / math.sqrt(_DH))
    smax = jnp.max(scores, axis=1, keepdims=True)
    p = jnp.exp(scores - smax)
    attn = p / jnp.sum(p, axis=1, keepdims=True)
    out_top = lax.dot_general(attn, V, (((1,), (0,)), ((), ())),
                              preferred_element_type=jnp.float32,
                              precision=_PREC_HI)
    meanV = jnp.mean(V, axis=0, keepdims=True)
    scat = lax.dot_general(onehot, out_top, (((0,), (0,)), ((), ())),
                           preferred_element_type=jnp.float32,
                           precision=_PREC_HI)
    rowsel = lax.dot_general(onehot, jnp.ones((_NTOP, 1), jnp.float32),
                             (((0,), (0,)), ((), ())),
                             preferred_element_type=jnp.float32,
                             precision=_PREC_HI)
    return scat + (1.0 - rowsel) * meanV


def _attn(qkv, counts):
    hp = _H // 2
    return pl.pallas_call(
        _attn_body,
        grid=(hp,),
        in_specs=[
            pl.BlockSpec((_L, _L), lambda h: (0, 0)),
            pl.BlockSpec((_L, 2 * _DH), lambda h: (0, h)),
            pl.BlockSpec((_L, 2 * _DH), lambda h: (0, hp + h)),
            pl.BlockSpec((_L, 2 * _DH), lambda h: (0, 2 * hp + h)),
        ],
        out_specs=pl.BlockSpec((_L, 2 * _DH), lambda h: (0, h)),
        out_shape=jax.ShapeDtypeStruct((_L, _D), jnp.float32),
    )(counts, qkv, qkv, qkv)


def kernel(x, Wq, bq, Wk, bk, Wv, bv, Wo, bo, W1, b1, W2, b2,
           g1, be1, g2, be2, gF, bF):
    xb = x[0]
    for i in range(_E_LAYERS):
        wqkv = jnp.concatenate([Wq[i], Wk[i], Wv[i]], axis=1)
        bqkv = jnp.concatenate([bq[i], bk[i], bv[i]])
        qkv = _mm_bias(xb, wqkv, bqkv)
        ctx = _attn(qkv, jnp.asarray(_COUNTS[i]))
        xb = _mm_res_ln(ctx, Wo[i], bo[i], xb, g1[i], be1[i])
        h = _mm_bias(xb, W1[i], b1[i], relu=True)
        final = (gF, bF) if i == _E_LAYERS - 1 else None
        xb = _mm_res_ln(h, W2[i], b2[i], xb, g2[i], be2[i], final=final)
    return xb[None]


# speedup vs baseline: 2.2791x; 1.7475x over previous
import functools
import math

import numpy as np
import jax
import jax.numpy as jnp
from jax import lax
from jax.experimental import pallas as pl
from jax.experimental.pallas import tpu as pltpu

_E_LAYERS = 2
_D = 1024
_H = 16
_DH = 64
_DFF = 4096
_L = 2048
_FACTOR = 5
_SAMPLE_K = min(_FACTOR * int(math.ceil(math.log(_L))), _L)
_NTOP = min(_FACTOR * int(math.ceil(math.log(_L))), _L)

_PREC_HI = lax.Precision.HIGHEST
_PREC_DEF = lax.Precision.DEFAULT


def _rotl32(x, r):
    return ((x << np.uint32(r)) | (x >> np.uint32(32 - r))).astype(np.uint32)


def _threefry2x32(k0, k1, x0, x1):
    rot_a = (13, 15, 26, 6)
    rot_b = (17, 29, 16, 24)
    ks0 = np.uint32(k0)
    ks1 = np.uint32(k1)
    ks2 = np.uint32(ks0 ^ ks1 ^ np.uint32(0x1BD11BDA))
    x0 = (x0 + ks0).astype(np.uint32)
    x1 = (x1 + ks1).astype(np.uint32)
    ks = (ks0, ks1, ks2)
    for g in range(5):
        for r in rot_a if g % 2 == 0 else rot_b:
            x0 = (x0 + x1).astype(np.uint32)
            x1 = _rotl32(x1, r)
            x1 = (x1 ^ x0).astype(np.uint32)
        x0 = (x0 + ks[(g + 1) % 3]).astype(np.uint32)
        x1 = (x1 + ks[(g + 2) % 3] + np.uint32(g + 1)).astype(np.uint32)
    return x0, x1


def _sample_idx(layer: int) -> np.ndarray:
    seed = 1234 + layer
    b1, b2 = _threefry2x32(np.uint32(0), np.uint32(seed),
                           np.zeros(2, np.uint32),
                           np.arange(2, dtype=np.uint32))
    size = _L * _SAMPLE_K
    h, lo = _threefry2x32(b1[1], b2[1], np.zeros(size, np.uint32),
                          np.arange(size, dtype=np.uint32))
    bits = (h ^ lo).astype(np.uint32)
    return (bits % np.uint32(_L)).astype(np.int32).reshape(_L, _SAMPLE_K)


def _count_matrix(layer: int) -> np.ndarray:
    idx = _sample_idx(layer)
    c = np.zeros((_L, _L), np.int8)
    np.add.at(c, (np.arange(_L)[:, None], idx), 1)
    return c


_COUNTS = [_count_matrix(i) for i in range(_E_LAYERS)]


def _ln(z, g, b):
    m = jnp.mean(z, axis=-1, keepdims=True)
    c = z - m
    v = jnp.mean(c * c, axis=-1, keepdims=True)
    return c * lax.rsqrt(v + 1e-5) * g + b


def _mm_bias_body(x_ref, w_ref, b_ref, o_ref, *, relu):
    acc = jnp.dot(x_ref[...], w_ref[...], preferred_element_type=jnp.float32,
                  precision=_PREC_DEF)
    acc = acc + b_ref[...]
    if relu:
        acc = jnp.maximum(acc, 0.0)
    o_ref[...] = acc


def _mm_bias(x, w, b, relu=False, bm=512, bn=1024):
    m, k = x.shape
    n = w.shape[1]
    body = functools.partial(_mm_bias_body, relu=relu)
    return pl.pallas_call(
        body,
        grid=(n // bn, m // bm),
        in_specs=[
            pl.BlockSpec((bm, k), lambda j, i: (i, 0)),
            pl.BlockSpec((k, bn), lambda j, i: (0, j)),
            pl.BlockSpec((1, bn), lambda j, i: (0, j)),
        ],
        out_specs=pl.BlockSpec((bm, bn), lambda j, i: (i, j)),
        out_shape=jax.ShapeDtypeStruct((m, n), jnp.float32),
    )(x, w, b.reshape(1, n))


def _mm_res_ln_body(y_ref, w_ref, b_ref, r_ref, g_ref, be_ref, o_ref,
                    acc_ref, *, nk):
    kk = pl.program_id(1)

    @pl.when(kk == 0)
    def _():
        acc_ref[...] = jnp.zeros_like(acc_ref)

    acc_ref[...] += jnp.dot(y_ref[...], w_ref[...],
                            preferred_element_type=jnp.float32,
                            precision=_PREC_DEF)

    @pl.when(kk == nk - 1)
    def _():
        z = acc_ref[...] + b_ref[...] + r_ref[...]
        o_ref[...] = _ln(z, g_ref[...], be_ref[...])


def _mm_res_ln(y, w, b, res, g, be, final=None, bm=512, bk=1024):
    m, k = y.shape
    n = w.shape[1]
    nk = k // bk
    if final is not None:
        gf, bf = final
        body = functools.partial(_mm_res_ln_final_body, nk=nk)
        args = [y, w, b.reshape(1, n), res, g.reshape(1, n),
                be.reshape(1, n), gf.reshape(1, n), bf.reshape(1, n)]
        extra = [pl.BlockSpec((1, n), lambda i, kk: (0, 0)),
                 pl.BlockSpec((1, n), lambda i, kk: (0, 0))]
    else:
        body = functools.partial(_mm_res_ln_body, nk=nk)
        args = [y, w, b.reshape(1, n), res, g.reshape(1, n), be.reshape(1, n)]
        extra = []
    in_specs = [
        pl.BlockSpec((bm, bk), lambda i, kk: (i, kk)),
        pl.BlockSpec((bk, n), lambda i, kk: (kk, 0)),
        pl.BlockSpec((1, n), lambda i, kk: (0, 0)),
        pl.BlockSpec((bm, n), lambda i, kk: (i, 0)),
        pl.BlockSpec((1, n), lambda i, kk: (0, 0)),
        pl.BlockSpec((1, n), lambda i, kk: (0, 0)),
    ] + extra
    return pl.pallas_call(
        body,
        grid=(m // bm, nk),
        in_specs=in_specs,
        out_specs=pl.BlockSpec((bm, n), lambda i, kk: (i, 0)),
        out_shape=jax.ShapeDtypeStruct((m, n), jnp.float32),
        scratch_shapes=[pltpu.VMEM((bm, n), jnp.float32)],
    )(*args)


def _mm_res_ln_final_body(y_ref, w_ref, b_ref, r_ref, g_ref, be_ref,
                          gf_ref, bf_ref, o_ref, acc_ref, *, nk):
    kk = pl.program_id(1)

    @pl.when(kk == 0)
    def _():
        acc_ref[...] = jnp.zeros_like(acc_ref)

    acc_ref[...] += jnp.dot(y_ref[...], w_ref[...],
                            preferred_element_type=jnp.float32,
                            precision=_PREC_DEF)

    @pl.when(kk == nk - 1)
    def _():
        z = acc_ref[...] + b_ref[...] + r_ref[...]
        z = _ln(z, g_ref[...], be_ref[...])
        o_ref[...] = _ln(z, gf_ref[...], bf_ref[...])


def _attn_body(c_ref, q_ref, k_ref, v_ref, o_ref):
    for s in range(2):
        sl = slice(s * _DH, (s + 1) * _DH)
        o_ref[:, sl] = _one_head(c_ref, q_ref[:, sl], k_ref[:, sl],
                                 v_ref[:, sl])


def _one_head(c_ref, Q, K, V):
    Kb = K.astype(jnp.bfloat16)
    iota_l = lax.broadcasted_iota(jnp.int32, (1, _L), 1)

    chunks = []
    cs = _L // 4
    for t in range(4):
        qt = Q[t * cs:(t + 1) * cs]
        qk = lax.dot_general(qt.astype(jnp.bfloat16), Kb,
                             (((1,), (1,)), ((), ())),
                             preferred_element_type=jnp.float32)
        ct = c_ref[t * cs:(t + 1) * cs].astype(jnp.float32)
        mx = jnp.max(jnp.where(ct > 0.0, qk, -jnp.inf), axis=1)
        sm = jnp.sum(qk * ct, axis=1) * (1.0 / _SAMPLE_K)
        chunks.append((mx - sm).reshape(1, cs))
    M = jnp.concatenate(chunks, axis=1)

    sub_iota = lax.broadcasted_iota(jnp.int32, (_NTOP, 1), 0)

    def step(i, carry):
        Mc, oh = carry
        amax = jnp.max(Mc)
        idx = jnp.min(jnp.where(Mc == amax, iota_l, _L))
        oh = jnp.where((sub_iota == i) & (iota_l == idx), 1.0, oh)
        Mc = jnp.where(iota_l == idx, -jnp.inf, Mc)
        return Mc, oh

    _, onehot = lax.fori_loop(
        0, _NTOP, step, (M, jnp.zeros((_NTOP, _L), jnp.float32)))

    Qr = lax.dot_general(onehot, Q, (((1,), (0,)), ((), ())),
                         preferred_element_type=jnp.float32,
                         precision=_PREC_HI)
    scores = lax.dot_general(Qr, K, (((1,), (1,)), ((), ())),
                             preferred_element_type=jnp.float32,
                             precision=_PREC_DEF) * (1.0 / math.sqrt(_DH))
    smax = jnp.max(scores, axis=1, keepdims=True)
    p = jnp.exp(scores - smax)
    attn = p / jnp.sum(p, axis=1, keepdims=True)
    out_top = lax.dot_general(attn, V, (((1,), (0,)), ((), ())),
                              preferred_element_type=jnp.float32,
                              precision=_PREC_DEF)
    meanV = jnp.mean(V, axis=0, keepdims=True)
    scat = lax.dot_general(onehot, out_top, (((0,), (0,)), ((), ())),
                           preferred_element_type=jnp.float32,
                           precision=_PREC_HI)
    rowsel = lax.dot_general(onehot, jnp.ones((_NTOP, 1), jnp.float32),
                             (((0,), (0,)), ((), ())),
                             preferred_element_type=jnp.float32,
                             precision=_PREC_HI)
    return scat + (1.0 - rowsel) * meanV


def _attn(qkv, counts):
    hp = _H // 2
    return pl.pallas_call(
        _attn_body,
        grid=(hp,),
        in_specs=[
            pl.BlockSpec((_L, _L), lambda h: (0, 0)),
            pl.BlockSpec((_L, 2 * _DH), lambda h: (0, h)),
            pl.BlockSpec((_L, 2 * _DH), lambda h: (0, hp + h)),
            pl.BlockSpec((_L, 2 * _DH), lambda h: (0, 2 * hp + h)),
        ],
        out_specs=pl.BlockSpec((_L, 2 * _DH), lambda h: (0, h)),
        out_shape=jax.ShapeDtypeStruct((_L, _D), jnp.float32),
    )(counts, qkv, qkv, qkv)


def kernel(x, Wq, bq, Wk, bk, Wv, bv, Wo, bo, W1, b1, W2, b2,
           g1, be1, g2, be2, gF, bF):
    xb = x[0]
    for i in range(_E_LAYERS):
        wqkv = jnp.concatenate([Wq[i], Wk[i], Wv[i]], axis=1)
        bqkv = jnp.concatenate([bq[i], bk[i], bv[i]])
        qkv = _mm_bias(xb, wqkv, bqkv)
        ctx = _attn(qkv, jnp.asarray(_COUNTS[i]))
        xb = _mm_res_ln(ctx, Wo[i], bo[i], xb, g1[i], be1[i])
        h = _mm_bias(xb, W1[i], b1[i], relu=True)
        final = (gF, bF) if i == _E_LAYERS - 1 else None
        xb = _mm_res_ln(h, W2[i], b2[i], xb, g2[i], be2[i], final=final)
    return xb[None]


# speedup vs baseline: 3.0204x; 1.3253x over previous
import functools
import math

import numpy as np
import jax
import jax.numpy as jnp
from jax import lax
from jax.experimental import pallas as pl
from jax.experimental.pallas import tpu as pltpu

_E_LAYERS = 2
_D = 1024
_H = 16
_DH = 64
_DFF = 4096
_L = 2048
_FACTOR = 5
_SAMPLE_K = min(_FACTOR * int(math.ceil(math.log(_L))), _L)
_NTOP = min(_FACTOR * int(math.ceil(math.log(_L))), _L)

_PREC_HI = lax.Precision.HIGHEST
_PREC_DEF = lax.Precision.DEFAULT
_HP = _H // 2
_CS = _L // 8
_FJ = 4
_FB = _DFF // _FJ


def _rotl32(x, r):
    return ((x << np.uint32(r)) | (x >> np.uint32(32 - r))).astype(np.uint32)


def _threefry2x32(k0, k1, x0, x1):
    rot_a = (13, 15, 26, 6)
    rot_b = (17, 29, 16, 24)
    ks0 = np.uint32(k0)
    ks1 = np.uint32(k1)
    ks2 = np.uint32(ks0 ^ ks1 ^ np.uint32(0x1BD11BDA))
    x0 = (x0 + ks0).astype(np.uint32)
    x1 = (x1 + ks1).astype(np.uint32)
    ks = (ks0, ks1, ks2)
    for g in range(5):
        for r in rot_a if g % 2 == 0 else rot_b:
            x0 = (x0 + x1).astype(np.uint32)
            x1 = _rotl32(x1, r)
            x1 = (x1 ^ x0).astype(np.uint32)
        x0 = (x0 + ks[(g + 1) % 3]).astype(np.uint32)
        x1 = (x1 + ks[(g + 2) % 3] + np.uint32(g + 1)).astype(np.uint32)
    return x0, x1


def _sample_idx(layer: int) -> np.ndarray:
    seed = 1234 + layer
    b1, b2 = _threefry2x32(np.uint32(0), np.uint32(seed),
                           np.zeros(2, np.uint32),
                           np.arange(2, dtype=np.uint32))
    size = _L * _SAMPLE_K
    h, lo = _threefry2x32(b1[1], b2[1], np.zeros(size, np.uint32),
                          np.arange(size, dtype=np.uint32))
    bits = (h ^ lo).astype(np.uint32)
    return (bits % np.uint32(_L)).astype(np.int32).reshape(_L, _SAMPLE_K)


def _count_matrix(layer: int) -> np.ndarray:
    idx = _sample_idx(layer)
    c = np.zeros((_L, _L), np.float32)
    np.add.at(c, (np.arange(_L)[:, None], idx), 1.0)
    return c.astype(np.float32)


_COUNTS = [_count_matrix(i).astype(jnp.bfloat16) for i in range(_E_LAYERS)]


def _ln(z, g, b):
    m = jnp.mean(z, axis=-1, keepdims=True)
    c = z - m
    v = jnp.mean(c * c, axis=-1, keepdims=True)
    return c * lax.rsqrt(v + 1e-5) * g + b


def _one_head(cm_ref, Q, K, V, idx_ref):
    Kb = K.astype(jnp.bfloat16)
    iota_l = lax.broadcasted_iota(jnp.int32, (1, _L), 1)

    chunks = []
    for t in range(_L // _CS):
        qt = Q[t * _CS:(t + 1) * _CS]
        qk = lax.dot_general(qt.astype(jnp.bfloat16), Kb,
                             (((1,), (1,)), ((), ())),
                             preferred_element_type=jnp.float32)
        ct = cm_ref[t * _CS:(t + 1) * _CS, :]
        ctf = ct.astype(jnp.float32)
        mx = jnp.max(jnp.where(ctf > 0.0, qk, -jnp.inf), axis=1)
        sm = jnp.sum(qk * ctf, axis=1) * (1.0 / _SAMPLE_K)
        chunks.append((mx - sm).reshape(1, _CS))
    M = jnp.concatenate(chunks, axis=1)

    def step(i, Mc):
        amax = jnp.max(Mc)
        idx = jnp.min(jnp.where(Mc == amax, iota_l, _L))
        idx_ref[pl.ds(i, 1), :] = jnp.full((1, 1), idx, jnp.int32)
        return jnp.where(iota_l == idx, -jnp.inf, Mc)

    lax.fori_loop(0, _NTOP, step, M)
    onehot = (idx_ref[...] == iota_l).astype(jnp.float32)

    Qr = lax.dot_general(onehot, Q, (((1,), (0,)), ((), ())),
                         preferred_element_type=jnp.float32,
                         precision=_PREC_HI)
    scores = lax.dot_general(Qr, K, (((1,), (1,)), ((), ())),
                             preferred_element_type=jnp.float32,
                             precision=_PREC_DEF) * (1.0 / math.sqrt(_DH))
    smax = jnp.max(scores, axis=1, keepdims=True)
    p = jnp.exp(scores - smax)
    attn = p / jnp.sum(p, axis=1, keepdims=True)
    out_top = lax.dot_general(attn, V, (((1,), (0,)), ((), ())),
                              preferred_element_type=jnp.float32,
                              precision=_PREC_DEF)
    meanV = jnp.mean(V, axis=0, keepdims=True)
    scat = lax.dot_general(onehot, out_top, (((0,), (0,)), ((), ())),
                           preferred_element_type=jnp.float32,
                           precision=_PREC_HI)
    rowsel = lax.dot_general(onehot, jnp.ones((_NTOP, 1), jnp.float32),
                             (((0,), (0,)), ((), ())),
                             preferred_element_type=jnp.float32,
                             precision=_PREC_HI)
    return scat + (1.0 - rowsel) * meanV


def _attn_layer_body(x_ref, cm_ref, wq_ref, wk_ref, wv_ref, bq_ref, bk_ref,
                     bv_ref, wo_ref, bo_ref, g_ref, be_ref, o_ref,
                     acc_ref, idx_ref):
    hp = pl.program_id(0)
    X = x_ref[...]
    q2 = jnp.dot(X, wq_ref[0], preferred_element_type=jnp.float32,
                 precision=_PREC_DEF) + bq_ref[0]
    k2 = jnp.dot(X, wk_ref[0], preferred_element_type=jnp.float32,
                 precision=_PREC_DEF) + bk_ref[0]
    v2 = jnp.dot(X, wv_ref[0], preferred_element_type=jnp.float32,
                 precision=_PREC_DEF) + bv_ref[0]
    ctxs = []
    for s in range(2):
        sl = slice(s * _DH, (s + 1) * _DH)
        ctxs.append(_one_head(cm_ref, q2[:, sl], k2[:, sl], v2[:, sl],
                              idx_ref))
    ctx = jnp.concatenate(ctxs, axis=1)

    @pl.when(hp == 0)
    def _():
        acc_ref[...] = jnp.zeros_like(acc_ref)

    acc_ref[...] += jnp.dot(ctx, wo_ref[0], preferred_element_type=jnp.float32,
                            precision=_PREC_DEF)

    @pl.when(hp == _HP - 1)
    def _():
        for t in range(4):
            rs = slice(t * (_L // 4), (t + 1) * (_L // 4))
            z = acc_ref[rs, :] + bo_ref[0] + x_ref[rs, :]
            o_ref[rs, :] = _ln(z, g_ref[0], be_ref[0])


def _attn_layer(x, cm, wq, wk, wv, bq, bk, bv, wo, bo, g1, be1, layer):
    i = layer
    full = pl.BlockSpec((_L, _D), lambda h: (0, 0))
    colw = pl.BlockSpec((1, _D, 2 * _DH), lambda h: (i, 0, h))
    colb = pl.BlockSpec((1, 1, 2 * _DH), lambda h: (i, 0, h))
    vec = pl.BlockSpec((1, 1, _D), lambda h: (i, 0, 0))
    return pl.pallas_call(
        _attn_layer_body,
        grid=(_HP,),
        in_specs=[
            full,
            pl.BlockSpec((_L, _L), lambda h: (0, 0)),
            colw, colw, colw,
            colb, colb, colb,
            pl.BlockSpec((1, 2 * _DH, _D), lambda h: (i, h, 0)),
            vec, vec, vec,
        ],
        out_specs=full,
        out_shape=jax.ShapeDtypeStruct((_L, _D), jnp.float32),
        scratch_shapes=[pltpu.VMEM((_L, _D), jnp.float32),
                        pltpu.VMEM((_NTOP, 1), jnp.int32)],
    )(x, cm, wq, wk, wv, bq, bk, bv, wo, bo, g1, be1)


def _ffn_body(x_ref, w1_ref, b1_ref, w2_ref, b2_ref, g_ref, be_ref,
              gf_ref, bf_ref, o_ref, acc_ref, *, final):
    j = pl.program_id(0)
    X = x_ref[...]
    h = jnp.dot(X, w1_ref[0], preferred_element_type=jnp.float32,
                precision=_PREC_DEF) + b1_ref[0]
    h = jnp.maximum(h, 0.0)

    @pl.when(j == 0)
    def _():
        acc_ref[...] = jnp.zeros_like(acc_ref)

    acc_ref[...] += jnp.dot(h, w2_ref[0], preferred_element_type=jnp.float32,
                            precision=_PREC_DEF)

    @pl.when(j == _FJ - 1)
    def _():
        for t in range(4):
            rs = slice(t * (_L // 4), (t + 1) * (_L // 4))
            z = acc_ref[rs, :] + b2_ref[0] + x_ref[rs, :]
            z = _ln(z, g_ref[0], be_ref[0])
            if final:
                z = _ln(z, gf_ref[0], bf_ref[0])
            o_ref[rs, :] = z


def _ffn_layer(x, w1, b1, w2, b2, g2, be2, gf, bf, layer, final):
    i = layer
    body = functools.partial(_ffn_body, final=final)
    vec = pl.BlockSpec((1, 1, _D), lambda j: (i, 0, 0))
    vecf = pl.BlockSpec((1, 1, _D), lambda j: (0, 0, 0))
    return pl.pallas_call(
        body,
        grid=(_FJ,),
        in_specs=[
            pl.BlockSpec((_L, _D), lambda j: (0, 0)),
            pl.BlockSpec((1, _D, _FB), lambda j: (i, 0, j)),
            pl.BlockSpec((1, 1, _FB), lambda j: (i, 0, j)),
            pl.BlockSpec((1, _FB, _D), lambda j: (i, j, 0)),
            vec, vec, vec,
            vecf, vecf,
        ],
        out_specs=pl.BlockSpec((_L, _D), lambda j: (0, 0)),
        out_shape=jax.ShapeDtypeStruct((_L, _D), jnp.float32),
        scratch_shapes=[pltpu.VMEM((_L, _D), jnp.float32)],
    )(x, w1, b1, w2, b2, g2, be2, gf, bf)


def kernel(x, Wq, bq, Wk, bk, Wv, bv, Wo, bo, W1, b1, W2, b2,
           g1, be1, g2, be2, gF, bF):
    xb = x[0]
    bq3 = bq.reshape(_E_LAYERS, 1, _D)
    bk3 = bk.reshape(_E_LAYERS, 1, _D)
    bv3 = bv.reshape(_E_LAYERS, 1, _D)
    bo3 = bo.reshape(_E_LAYERS, 1, _D)
    b13 = b1.reshape(_E_LAYERS, 1, _DFF)
    b23 = b2.reshape(_E_LAYERS, 1, _D)
    g13 = g1.reshape(_E_LAYERS, 1, _D)
    be13 = be1.reshape(_E_LAYERS, 1, _D)
    g23 = g2.reshape(_E_LAYERS, 1, _D)
    be23 = be2.reshape(_E_LAYERS, 1, _D)
    gf3 = gF.reshape(1, 1, _D)
    bf3 = bF.reshape(1, 1, _D)
    for i in range(_E_LAYERS):
        cm = jnp.asarray(_COUNTS[i])
        xb = _attn_layer(xb, cm, Wq, Wk, Wv, bq3, bk3, bv3, Wo, bo3,
                         g13, be13, layer=i)
        xb = _ffn_layer(xb, W1, b13, W2, b23, g23, be23, gf3, bf3,
                        layer=i, final=(i == _E_LAYERS - 1))
    return xb[None]


# speedup vs baseline: 3.1079x; 1.0290x over previous
import functools
import math

import numpy as np
import jax
import jax.numpy as jnp
from jax import lax
from jax.experimental import pallas as pl
from jax.experimental.pallas import tpu as pltpu

_E_LAYERS = 2
_D = 1024
_H = 16
_DH = 64
_DFF = 4096
_L = 2048
_FACTOR = 5
_SAMPLE_K = min(_FACTOR * int(math.ceil(math.log(_L))), _L)
_NTOP = min(_FACTOR * int(math.ceil(math.log(_L))), _L)

_PREC_HI = lax.Precision.HIGHEST
_PREC_DEF = lax.Precision.DEFAULT
_HP = _H // 2
_CS = _L // 8
_FJ = 4
_FB = _DFF // _FJ


def _rotl32(x, r):
    return ((x << np.uint32(r)) | (x >> np.uint32(32 - r))).astype(np.uint32)


def _threefry2x32(k0, k1, x0, x1):
    rot_a = (13, 15, 26, 6)
    rot_b = (17, 29, 16, 24)
    ks0 = np.uint32(k0)
    ks1 = np.uint32(k1)
    ks2 = np.uint32(ks0 ^ ks1 ^ np.uint32(0x1BD11BDA))
    x0 = (x0 + ks0).astype(np.uint32)
    x1 = (x1 + ks1).astype(np.uint32)
    ks = (ks0, ks1, ks2)
    for g in range(5):
        for r in rot_a if g % 2 == 0 else rot_b:
            x0 = (x0 + x1).astype(np.uint32)
            x1 = _rotl32(x1, r)
            x1 = (x1 ^ x0).astype(np.uint32)
        x0 = (x0 + ks[(g + 1) % 3]).astype(np.uint32)
        x1 = (x1 + ks[(g + 2) % 3] + np.uint32(g + 1)).astype(np.uint32)
    return x0, x1


def _sample_idx(layer: int) -> np.ndarray:
    seed = 1234 + layer
    b1, b2 = _threefry2x32(np.uint32(0), np.uint32(seed),
                           np.zeros(2, np.uint32),
                           np.arange(2, dtype=np.uint32))
    size = _L * _SAMPLE_K
    h, lo = _threefry2x32(b1[1], b2[1], np.zeros(size, np.uint32),
                          np.arange(size, dtype=np.uint32))
    bits = (h ^ lo).astype(np.uint32)
    return (bits % np.uint32(_L)).astype(np.int32).reshape(_L, _SAMPLE_K)


def _count_matrix(layer: int) -> np.ndarray:
    idx = _sample_idx(layer)
    c = np.zeros((_L, _L), np.float32)
    np.add.at(c, (np.arange(_L)[:, None], idx), 1.0)
    return c.astype(np.float32)


_COUNTS = [_count_matrix(i).astype(jnp.bfloat16) for i in range(_E_LAYERS)]


def _ln(z, g, b):
    m = jnp.mean(z, axis=-1, keepdims=True)
    c = z - m
    v = jnp.mean(c * c, axis=-1, keepdims=True)
    return c * lax.rsqrt(v + 1e-5) * g + b


def _one_head(cm_ref, Q, K, V, idx_ref):
    Kb = K.astype(jnp.bfloat16)
    iota_l = lax.broadcasted_iota(jnp.int32, (1, _L), 1)

    chunks = []
    for t in range(_L // _CS):
        qt = Q[t * _CS:(t + 1) * _CS]
        qk = lax.dot_general(qt.astype(jnp.bfloat16), Kb,
                             (((1,), (1,)), ((), ())),
                             preferred_element_type=jnp.float32)
        ct = cm_ref[t * _CS:(t + 1) * _CS, :]
        ctf = ct.astype(jnp.float32)
        mx = jnp.max(jnp.where(ctf > 0.0, qk, -jnp.inf), axis=1)
        sm = jnp.sum(qk * ctf, axis=1) * (1.0 / _SAMPLE_K)
        chunks.append((mx - sm).reshape(1, _CS))
    M = jnp.concatenate(chunks, axis=1)

    def step(i, Mc):
        amax = jnp.max(Mc)
        idx = jnp.min(jnp.where(Mc == amax, iota_l, _L))
        idx_ref[pl.ds(i, 1), :] = jnp.full((1, 1), idx, jnp.int32)
        return jnp.where(iota_l == idx, -jnp.inf, Mc)

    lax.fori_loop(0, _NTOP, step, M)
    onehot = (idx_ref[...] == iota_l).astype(jnp.float32)

    Qr = lax.dot_general(onehot, Q, (((1,), (0,)), ((), ())),
                         preferred_element_type=jnp.float32,
                         precision=_PREC_DEF)
    scores = lax.dot_general(Qr, K, (((1,), (1,)), ((), ())),
                             preferred_element_type=jnp.float32,
                             precision=_PREC_DEF) * (1.0 / math.sqrt(_DH))
    smax = jnp.max(scores, axis=1, keepdims=True)
    p = jnp.exp(scores - smax)
    attn = p / jnp.sum(p, axis=1, keepdims=True)
    out_top = lax.dot_general(attn, V, (((1,), (0,)), ((), ())),
                              preferred_element_type=jnp.float32,
                              precision=_PREC_DEF)
    meanV = jnp.mean(V, axis=0, keepdims=True)
    scat = lax.dot_general(onehot, out_top, (((0,), (0,)), ((), ())),
                           preferred_element_type=jnp.float32,
                           precision=_PREC_DEF)
    rowsel = lax.dot_general(onehot, jnp.ones((_NTOP, 1), jnp.float32),
                             (((0,), (0,)), ((), ())),
                             preferred_element_type=jnp.float32,
                             precision=_PREC_DEF)
    return scat + (1.0 - rowsel) * meanV


def _attn_layer_body(x_ref, cm_ref, wq_ref, wk_ref, wv_ref, bq_ref, bk_ref,
                     bv_ref, wo_ref, bo_ref, g_ref, be_ref, o_ref,
                     acc_ref, idx_ref):
    hp = pl.program_id(0)
    X = x_ref[...]
    q2 = jnp.dot(X, wq_ref[0], preferred_element_type=jnp.float32,
                 precision=_PREC_DEF) + bq_ref[0]
    k2 = jnp.dot(X, wk_ref[0], preferred_element_type=jnp.float32,
                 precision=_PREC_DEF) + bk_ref[0]
    v2 = jnp.dot(X, wv_ref[0], preferred_element_type=jnp.float32,
                 precision=_PREC_DEF) + bv_ref[0]
    ctxs = []
    for s in range(2):
        sl = slice(s * _DH, (s + 1) * _DH)
        ctxs.append(_one_head(cm_ref, q2[:, sl], k2[:, sl], v2[:, sl],
                              idx_ref))
    ctx = jnp.concatenate(ctxs, axis=1)

    @pl.when(hp == 0)
    def _():
        acc_ref[...] = jnp.zeros_like(acc_ref)

    acc_ref[...] += jnp.dot(ctx, wo_ref[0], preferred_element_type=jnp.float32,
                            precision=_PREC_DEF)

    @pl.when(hp == _HP - 1)
    def _():
        for t in range(4):
            rs = slice(t * (_L // 4), (t + 1) * (_L // 4))
            z = acc_ref[rs, :] + bo_ref[0] + x_ref[rs, :]
            o_ref[rs, :] = _ln(z, g_ref[0], be_ref[0])


def _attn_layer(x, cm, wq, wk, wv, bq, bk, bv, wo, bo, g1, be1, layer):
    i = layer
    full = pl.BlockSpec((_L, _D), lambda h: (0, 0))
    colw = pl.BlockSpec((1, _D, 2 * _DH), lambda h: (i, 0, h))
    colb = pl.BlockSpec((1, 1, 2 * _DH), lambda h: (i, 0, h))
    vec = pl.BlockSpec((1, 1, _D), lambda h: (i, 0, 0))
    return pl.pallas_call(
        _attn_layer_body,
        grid=(_HP,),
        in_specs=[
            full,
            pl.BlockSpec((_L, _L), lambda h: (0, 0)),
            colw, colw, colw,
            colb, colb, colb,
            pl.BlockSpec((1, 2 * _DH, _D), lambda h: (i, h, 0)),
            vec, vec, vec,
        ],
        out_specs=full,
        out_shape=jax.ShapeDtypeStruct((_L, _D), jnp.float32),
        scratch_shapes=[pltpu.VMEM((_L, _D), jnp.float32),
                        pltpu.VMEM((_NTOP, 1), jnp.int32)],
    )(x, cm, wq, wk, wv, bq, bk, bv, wo, bo, g1, be1)


def _ffn_body(x_ref, w1_ref, b1_ref, w2_ref, b2_ref, g_ref, be_ref,
              gf_ref, bf_ref, o_ref, acc_ref, *, final):
    j = pl.program_id(0)
    X = x_ref[...]
    h = jnp.dot(X.astype(jnp.bfloat16), w1_ref[0],
                preferred_element_type=jnp.float32,
                precision=_PREC_DEF) + b1_ref[0]
    h = jnp.maximum(h, 0.0)

    @pl.when(j == 0)
    def _():
        acc_ref[...] = jnp.zeros_like(acc_ref)

    acc_ref[...] += jnp.dot(h.astype(jnp.bfloat16), w2_ref[0],
                            preferred_element_type=jnp.float32,
                            precision=_PREC_DEF)

    @pl.when(j == _FJ - 1)
    def _():
        for t in range(4):
            rs = slice(t * (_L // 4), (t + 1) * (_L // 4))
            z = acc_ref[rs, :] + b2_ref[0] + x_ref[rs, :]
            z = _ln(z, g_ref[0], be_ref[0])
            if final:
                z = _ln(z, gf_ref[0], bf_ref[0])
            o_ref[rs, :] = z


def _ffn_layer(x, w1, b1, w2, b2, g2, be2, gf, bf, layer, final):
    i = layer
    body = functools.partial(_ffn_body, final=final)
    vec = pl.BlockSpec((1, 1, _D), lambda j: (i, 0, 0))
    vecf = pl.BlockSpec((1, 1, _D), lambda j: (0, 0, 0))
    return pl.pallas_call(
        body,
        grid=(_FJ,),
        in_specs=[
            pl.BlockSpec((_L, _D), lambda j: (0, 0)),
            pl.BlockSpec((1, _D, _FB), lambda j: (i, 0, j)),
            pl.BlockSpec((1, 1, _FB), lambda j: (i, 0, j)),
            pl.BlockSpec((1, _FB, _D), lambda j: (i, j, 0)),
            vec, vec, vec,
            vecf, vecf,
        ],
        out_specs=pl.BlockSpec((_L, _D), lambda j: (0, 0)),
        out_shape=jax.ShapeDtypeStruct((_L, _D), jnp.float32),
        scratch_shapes=[pltpu.VMEM((_L, _D), jnp.float32)],
    )(x, w1, b1, w2, b2, g2, be2, gf, bf)


def kernel(x, Wq, bq, Wk, bk, Wv, bv, Wo, bo, W1, b1, W2, b2,
           g1, be1, g2, be2, gF, bF):
    xb = x[0]
    bq3 = bq.reshape(_E_LAYERS, 1, _D)
    bk3 = bk.reshape(_E_LAYERS, 1, _D)
    bv3 = bv.reshape(_E_LAYERS, 1, _D)
    bo3 = bo.reshape(_E_LAYERS, 1, _D)
    b13 = b1.reshape(_E_LAYERS, 1, _DFF)
    b23 = b2.reshape(_E_LAYERS, 1, _D)
    g13 = g1.reshape(_E_LAYERS, 1, _D)
    be13 = be1.reshape(_E_LAYERS, 1, _D)
    g23 = g2.reshape(_E_LAYERS, 1, _D)
    be23 = be2.reshape(_E_LAYERS, 1, _D)
    gf3 = gF.reshape(1, 1, _D)
    bf3 = bF.reshape(1, 1, _D)
    W1b = W1.astype(jnp.bfloat16)
    W2b = W2.astype(jnp.bfloat16)
    for i in range(_E_LAYERS):
        cm = jnp.asarray(_COUNTS[i])
        xb = _attn_layer(xb, cm, Wq, Wk, Wv, bq3, bk3, bv3, Wo, bo3,
                         g13, be13, layer=i)
        xb = _ffn_layer(xb, W1b, b13, W2b, b23, g23, be23, gf3, bf3,
                        layer=i, final=(i == _E_LAYERS - 1))
    return xb[None]
